# baseline (speedup 1.0000x reference)
import functools

import jax
import jax.numpy as jnp
import numpy as np
from jax import lax
from jax.experimental import pallas as pl
from jax.experimental.pallas import tpu as pltpu
from jax.experimental.pallas import tpu_sc as plsc

_N_LEVELS = 20
_FPL = 2
_BASE_RES = 32
_SCALE = 1.5
_TABLE_SIZE = 1 << 21
_MASK = _TABLE_SIZE - 1
_RES = [int(np.floor(_BASE_RES * _SCALE**l)) for l in range(_N_LEVELS)]
_DENSE = [(r + 1) ** 3 <= _TABLE_SIZE for r in _RES]
_P2 = int(np.uint32(2654435761).astype(np.int64) - (1 << 32))
_P3 = 805459861
_CORNERS = [(i, j, k) for i in (0, 1) for j in (0, 1) for k in (0, 1)]

_NC = 2
_NS = 16
_NW = _NC * _NS
_L = 16

_R = _N_LEVELS * _TABLE_SIZE
_R8 = _R // 4

_C = 512
_NG = _C // _L
_NI = 8 * _C

_RB = 4096
_RPW = _TABLE_SIZE // _NW
_CPL = _RPW // _RB
_NIT = _N_LEVELS * _CPL

_SC_PARAMS = pltpu.CompilerParams(
    use_tc_tiling_on_sc=False, needs_layout_passes=False
)


def _inter_body(tabf, tabi, inb0, inb1, fb0, fb1, si0, si1, so0, so1):
  wid = lax.axis_index("s") * _NC + lax.axis_index("c")
  iota = lax.iota(jnp.int32, _L)

  def offs(it):
    l = it // _CPL
    q = it - l * _CPL
    row_local = wid * _RPW + q * _RB
    s0 = l * (2 * _TABLE_SIZE) + row_local
    d0 = (l * _TABLE_SIZE + row_local) // 4
    return s0, d0

  def start_in(it, inb, sem):
    s0, _ = offs(it)
    pltpu.async_copy(tabf.at[pl.ds(s0, _RB)], inb.at[0], sem)
    pltpu.async_copy(tabf.at[pl.ds(s0 + _TABLE_SIZE, _RB)], inb.at[1], sem)

  def wait_in(it, inb, sem):
    s0, _ = offs(it)
    pltpu.make_async_copy(tabf.at[pl.ds(s0, _RB)], inb.at[0], sem).wait()
    pltpu.make_async_copy(
        tabf.at[pl.ds(s0 + _TABLE_SIZE, _RB)], inb.at[1], sem
    ).wait()

  def do_chunk(it, inb, fb, sem):
    def g(gi, _):
      f0v = inb[0, pl.ds(gi * _L, _L)]
      f1v = inb[1, pl.ds(gi * _L, _L)]
      r = jnp.full((_L,), gi * _L, jnp.int32) + iota
      r8 = lax.shift_right_logical(r, 2)
      c0 = lax.shift_left(r & 3, 1)
      plsc.store_scatter(fb, [r8, c0], f0v)
      plsc.store_scatter(fb, [r8, c0 + 1], f1v)
      return 0

    lax.fori_loop(0, _RB // _L, g, 0)
    _, d0 = offs(it)
    pltpu.async_copy(fb, tabi.at[pl.ds(d0, _RB // 4)], sem)

  def wait_out(it, fb, sem):
    _, d0 = offs(it)
    pltpu.make_async_copy(fb, tabi.at[pl.ds(d0, _RB // 4)], sem).wait()

  start_in(0, inb0, si0)
  start_in(1, inb1, si1)

  def lp(it2, _):
    a = it2 * 2
    b = a + 1
    wait_in(a, inb0, si0)

    @pl.when(it2 > 0)
    def _():
      wait_out(a - 2, fb0, so0)

    do_chunk(a, inb0, fb0, so0)

    @pl.when(a + 2 < _NIT)
    def _():
      start_in(a + 2, inb0, si0)

    wait_in(b, inb1, si1)

    @pl.when(it2 > 0)
    def _():
      wait_out(b - 2, fb1, so1)

    do_chunk(b, inb1, fb1, so1)

    @pl.when(b + 2 < _NIT)
    def _():
      start_in(b + 2, inb1, si1)

    return 0

  lax.fori_loop(0, _NIT // 2, lp, 0)
  wait_out(_NIT - 2, fb0, so0)
  wait_out(_NIT - 1, fb1, so1)


def _interleave_sc(tabf):
  mesh = plsc.VectorSubcoreMesh(
      core_axis_name="c", subcore_axis_name="s", num_cores=_NC, num_subcores=_NS
  )
  f = pl.kernel(
      _inter_body,
      out_type=jax.ShapeDtypeStruct((_R8, 8), jnp.float32),
      mesh=mesh,
      scratch_types=[
          pltpu.VMEM((2, _RB), jnp.float32),
          pltpu.VMEM((2, _RB), jnp.float32),
          pltpu.VMEM((_RB // 4, 8), jnp.float32),
          pltpu.VMEM((_RB // 4, 8), jnp.float32),
          pltpu.SemaphoreType.DMA,
          pltpu.SemaphoreType.DMA,
          pltpu.SemaphoreType.DMA,
          pltpu.SemaphoreType.DMA,
      ],
      compiler_params=_SC_PARAMS,
  )
  return f(tabf)


def _encode_body(
    xT, tab, encT, pbuf, ib0, ib1, if0, if1, rb0, rb1, outt, sem0, sem1
):
  n = encT.shape[1]
  per_w = n // _NW
  nchunk = per_w // _C
  wid = lax.axis_index("s") * _NC + lax.axis_index("c")

  iota = lax.iota(jnp.int32, _L)

  def load_p(j):
    px = pbuf[0, pl.ds(j * _L, _L)]
    py = pbuf[1, pl.ds(j * _L, _L)]
    pz = pbuf[2, pl.ds(j * _L, _L)]
    return px, py, pz

  def compute_idx(l, ib, ifu):
    res = _RES[l]
    loff = l * _TABLE_SIZE

    def g(j, _):
      px, py, pz = load_p(j)
      x0 = (px * np.float32(res)).astype(jnp.int32)
      y0 = (py * np.float32(res)).astype(jnp.int32)
      z0 = (pz * np.float32(res)).astype(jnp.int32)
      if _DENSE[l]:
        s = res + 1
        a = x0 + y0 * np.int32(s) + z0 * np.int32(s * s) + np.int32(loff)
        rows = [a + np.int32(i + j2 * s + k * s * s) for (i, j2, k) in _CORNERS]
      else:
        u0 = x0
        u1 = x0 + np.int32(1)
        v0 = y0 * np.int32(_P2)
        v1 = v0 + np.int32(_P2)
        w0 = z0 * np.int32(_P3)
        w1 = w0 + np.int32(_P3)
        us, vs, ws = (u0, u1), (v0, v1), (w0, w1)
        rows = [
            ((us[i] ^ vs[j2] ^ ws[k]) & np.int32(_MASK)) + np.int32(loff)
            for (i, j2, k) in _CORNERS
        ]
      for c in range(8):
        ib[pl.ds(c * _C + j * _L, _L)] = lax.shift_right_logical(rows[c], 2)
        ifu[pl.ds(c * _C + j * _L, _L)] = rows[c]
      return 0

    lax.fori_loop(0, _NG, g, 0)

  def interp(l, rb, ifu):
    res = _RES[l]

    def g(j, _):
      px, py, pz = load_p(j)
      sx = px * np.float32(res)
      sy = py * np.float32(res)
      sz = pz * np.float32(res)
      fx = sx - sx.astype(jnp.int32).astype(jnp.float32)
      fy = sy - sy.astype(jnp.int32).astype(jnp.float32)
      fz = sz - sz.astype(jnp.int32).astype(jnp.float32)
      wx = (np.float32(1.0) - fx, fx)
      wy = (np.float32(1.0) - fy, fy)
      wz = (np.float32(1.0) - fz, fz)
      wxy = {(i, j2): wx[i] * wy[j2] for i in (0, 1) for j2 in (0, 1)}
      jb = jnp.full((_L,), j * _L, jnp.int32) + iota
      acc0 = jnp.zeros((_L,), jnp.float32)
      acc1 = jnp.zeros((_L,), jnp.float32)
      for c, (i, j2, k) in enumerate(_CORNERS):
        w = wxy[(i, j2)] * wz[k]
        i0 = jb + (c * _C)
        rv = ifu[pl.ds(c * _C + j * _L, _L)]
        c0 = lax.shift_left(rv & 3, 1)
        f0 = plsc.load_gather(rb, [i0, c0])
        f1 = plsc.load_gather(rb, [i0, c0 + 1])
        acc0 = acc0 + w * f0
        acc1 = acc1 + w * f1
      outt[2 * l, pl.ds(j * _L, _L)] = acc0
      outt[2 * l + 1, pl.ds(j * _L, _L)] = acc1
      return 0

    lax.fori_loop(0, _NG, g, 0)

  def chunk_body(ci, _):
    base = wid * per_w + ci * _C
    pltpu.sync_copy(xT.at[pl.ds(0, 3), pl.ds(base, _C)], pbuf)
    pltpu.sync_copy(xT.at[pl.ds(3, 3), pl.ds(base, _C)], outt.at[pl.ds(40, 3), :])

    ibs = (ib0, ib1)
    ifs = (if0, if1)
    rbs = (rb0, rb1)
    sems = (sem0, sem1)
    compute_idx(0, ibs[0], ifs[0])
    cps = [pltpu.async_copy(tab.at[ibs[0]], rbs[0], sems[0])]
    for l in range(1, _N_LEVELS):
      p = l & 1
      compute_idx(l, ibs[p], ifs[p])
      cps.append(pltpu.async_copy(tab.at[ibs[p]], rbs[p], sems[p]))
      cps[l - 1].wait()
      interp(l - 1, rbs[(l - 1) & 1], ifs[(l - 1) & 1])
    cps[_N_LEVELS - 1].wait()
    interp(_N_LEVELS - 1, rbs[(_N_LEVELS - 1) & 1], ifs[(_N_LEVELS - 1) & 1])

    pltpu.sync_copy(outt, encT.at[:, pl.ds(base, _C)])
    return 0

  lax.fori_loop(0, nchunk, chunk_body, 0)


def _encode_sc(xT, tab, n):
  mesh = plsc.VectorSubcoreMesh(
      core_axis_name="c", subcore_axis_name="s", num_cores=_NC, num_subcores=_NS
  )
  f = pl.kernel(
      _encode_body,
      out_type=jax.ShapeDtypeStruct((43, n), jnp.float32),
      mesh=mesh,
      scratch_types=[
          pltpu.VMEM((3, _C), jnp.float32),
          pltpu.VMEM((_NI,), jnp.int32),
          pltpu.VMEM((_NI,), jnp.int32),
          pltpu.VMEM((_NI,), jnp.int32),
          pltpu.VMEM((_NI,), jnp.int32),
          pltpu.VMEM((_NI, 8), jnp.float32),
          pltpu.VMEM((_NI, 8), jnp.float32),
          pltpu.VMEM((43, _C), jnp.float32),
          pltpu.SemaphoreType.DMA,
          pltpu.SemaphoreType.DMA,
      ],
      compiler_params=_SC_PARAMS,
  )
  return f(xT, tab)


def _mlp_body(enc_ref, w1t_ref, b1_ref, w2t_ref, b2_ref, out_ref):
  e = enc_ref[...]
  h = lax.dot_general(
      w1t_ref[...], e, (((1,), (0,)), ((), ())),
      preferred_element_type=jnp.float32,
  )
  h = jnp.maximum(h + b1_ref[...], np.float32(0.0))
  o = lax.dot_general(
      w2t_ref[...], h, (((1,), (0,)), ((), ())),
      preferred_element_type=jnp.float32,
  )
  out_ref[...] = o + b2_ref[...]


def _mlp_tc(encT, w1t, b1c, w2t, b2c, n):
  bn = 2048
  grid = (n // bn,)
  return pl.pallas_call(
      _mlp_body,
      grid=grid,
      in_specs=[
          pl.BlockSpec((43, bn), lambda i: (0, i)),
          pl.BlockSpec((64, 43), lambda i: (0, 0)),
          pl.BlockSpec((64, 1), lambda i: (0, 0)),
          pl.BlockSpec((1, 64), lambda i: (0, 0)),
          pl.BlockSpec((1, 1), lambda i: (0, 0)),
      ],
      out_specs=pl.BlockSpec((1, bn), lambda i: (0, i)),
      out_shape=jax.ShapeDtypeStruct((1, n), jnp.float32),
  )(encT, w1t, b1c, w2t, b2c)


@jax.jit
def kernel(x, table, W1, b1, W2, b2):
  n = x.shape[0]
  xT = x.T
  tabf = table.transpose(0, 2, 1).reshape(-1)
  tabi = _interleave_sc(tabf)
  encT = _encode_sc(xT, tabi, n)
  sdf = _mlp_tc(encT, W1.T, b1.reshape(64, 1), W2.T, b2.reshape(1, 1), n)
  return sdf[0]


# speedup vs baseline: 1.0013x; 1.0013x over previous
import jax
import jax.numpy as jnp
import numpy as np
from jax import lax
from jax.experimental import pallas as pl
from jax.experimental.pallas import tpu as pltpu
from jax.experimental.pallas import tpu_sc as plsc

_N_LEVELS = 20
_FPL = 2
_BASE_RES = 32
_SCALE = 1.5
_TABLE_SIZE = 1 << 21
_MASK = _TABLE_SIZE - 1
_RES = [int(np.floor(_BASE_RES * _SCALE**l)) for l in range(_N_LEVELS)]
_DENSE = [(r + 1) ** 3 <= _TABLE_SIZE for r in _RES]
_P2 = int(np.uint32(2654435761).astype(np.int64) - (1 << 32))
_P3 = 805459861
_CORNERS = [(i, j, k) for i in (0, 1) for j in (0, 1) for k in (0, 1)]

_NC = 2
_NS = 16
_NW = _NC * _NS
_L = 16

_R = _N_LEVELS * _TABLE_SIZE
_R8 = _R // 4

_C = 256
_NG = _C // _L
_NI = 8 * _C

_RB = 4096
_RPW = _TABLE_SIZE // _NW
_CPL = _RPW // _RB
_NIT = _N_LEVELS * _CPL

_SC_PARAMS = pltpu.CompilerParams(
    use_tc_tiling_on_sc=False, needs_layout_passes=False
)


def _inter_body(tabf, tabi, inb0, inb1, fb0, fb1, si0, si1, so0, so1):
  wid = lax.axis_index("s") * _NC + lax.axis_index("c")
  iota = lax.iota(jnp.int32, _L)

  def offs(it):
    l = it // _CPL
    q = it - l * _CPL
    row_local = wid * _RPW + q * _RB
    s0 = l * (2 * _TABLE_SIZE) + row_local
    d0 = (l * _TABLE_SIZE + row_local) // 4
    return s0, d0

  def start_in(it, inb, sem):
    s0, _ = offs(it)
    pltpu.async_copy(tabf.at[pl.ds(s0, _RB)], inb.at[0], sem)
    pltpu.async_copy(tabf.at[pl.ds(s0 + _TABLE_SIZE, _RB)], inb.at[1], sem)

  def wait_in(it, inb, sem):
    s0, _ = offs(it)
    pltpu.make_async_copy(tabf.at[pl.ds(s0, _RB)], inb.at[0], sem).wait()
    pltpu.make_async_copy(
        tabf.at[pl.ds(s0 + _TABLE_SIZE, _RB)], inb.at[1], sem
    ).wait()

  def do_chunk(it, inb, fb, sem):
    def g(gi, _):
      f0v = inb[0, pl.ds(gi * _L, _L)]
      f1v = inb[1, pl.ds(gi * _L, _L)]
      r = jnp.full((_L,), gi * _L, jnp.int32) + iota
      r8 = lax.shift_right_logical(r, 2)
      c0 = lax.shift_left(r & 3, 1)
      plsc.store_scatter(fb, [r8, c0], f0v)
      plsc.store_scatter(fb, [r8, c0 + 1], f1v)
      return 0

    lax.fori_loop(0, _RB // _L, g, 0)
    _, d0 = offs(it)
    pltpu.async_copy(fb, tabi.at[pl.ds(d0, _RB // 4)], sem)

  def wait_out(it, fb, sem):
    _, d0 = offs(it)
    pltpu.make_async_copy(fb, tabi.at[pl.ds(d0, _RB // 4)], sem).wait()

  start_in(0, inb0, si0)
  start_in(1, inb1, si1)

  def lp(it2, _):
    a = it2 * 2
    b = a + 1
    wait_in(a, inb0, si0)

    @pl.when(it2 > 0)
    def _():
      wait_out(a - 2, fb0, so0)

    do_chunk(a, inb0, fb0, so0)

    @pl.when(a + 2 < _NIT)
    def _():
      start_in(a + 2, inb0, si0)

    wait_in(b, inb1, si1)

    @pl.when(it2 > 0)
    def _():
      wait_out(b - 2, fb1, so1)

    do_chunk(b, inb1, fb1, so1)

    @pl.when(b + 2 < _NIT)
    def _():
      start_in(b + 2, inb1, si1)

    return 0

  lax.fori_loop(0, _NIT // 2, lp, 0)
  wait_out(_NIT - 2, fb0, so0)
  wait_out(_NIT - 1, fb1, so1)


def _interleave_sc(tabf):
  mesh = plsc.VectorSubcoreMesh(
      core_axis_name="c", subcore_axis_name="s", num_cores=_NC, num_subcores=_NS
  )
  f = pl.kernel(
      _inter_body,
      out_type=jax.ShapeDtypeStruct((_R8, 8), jnp.float32),
      mesh=mesh,
      scratch_types=[
          pltpu.VMEM((2, _RB), jnp.float32),
          pltpu.VMEM((2, _RB), jnp.float32),
          pltpu.VMEM((_RB // 4, 8), jnp.float32),
          pltpu.VMEM((_RB // 4, 8), jnp.float32),
          pltpu.SemaphoreType.DMA,
          pltpu.SemaphoreType.DMA,
          pltpu.SemaphoreType.DMA,
          pltpu.SemaphoreType.DMA,
      ],
      compiler_params=_SC_PARAMS,
  )
  return f(tabf)


def _encode_body(
    xT, tab, encT, pbuf, ib0, ib1, if0, if1, rb0, rb1, outt, sem0, sem1
):
  n = encT.shape[1]
  per_w = n // _NW
  nchunk = per_w // _C
  wid = lax.axis_index("s") * _NC + lax.axis_index("c")

  iota = lax.iota(jnp.int32, _L)

  def load_p(j):
    px = pbuf[0, pl.ds(j * _L, _L)]
    py = pbuf[1, pl.ds(j * _L, _L)]
    pz = pbuf[2, pl.ds(j * _L, _L)]
    return px, py, pz

  def compute_idx(l, ib, ifu):
    res = _RES[l]
    loff = l * _TABLE_SIZE

    def g(j, _):
      px, py, pz = load_p(j)
      x0 = (px * np.float32(res)).astype(jnp.int32)
      y0 = (py * np.float32(res)).astype(jnp.int32)
      z0 = (pz * np.float32(res)).astype(jnp.int32)
      if _DENSE[l]:
        s = res + 1
        a = x0 + y0 * np.int32(s) + z0 * np.int32(s * s) + np.int32(loff)
        rows = [a + np.int32(i + j2 * s + k * s * s) for (i, j2, k) in _CORNERS]
      else:
        u0 = x0
        u1 = x0 + np.int32(1)
        v0 = y0 * np.int32(_P2)
        v1 = v0 + np.int32(_P2)
        w0 = z0 * np.int32(_P3)
        w1 = w0 + np.int32(_P3)
        us, vs, ws = (u0, u1), (v0, v1), (w0, w1)
        rows = [
            ((us[i] ^ vs[j2] ^ ws[k]) & np.int32(_MASK)) + np.int32(loff)
            for (i, j2, k) in _CORNERS
        ]
      for c in range(8):
        ib[pl.ds(c * _C + j * _L, _L)] = lax.shift_right_logical(rows[c], 2)
        ifu[pl.ds(c * _C + j * _L, _L)] = rows[c]
      return 0

    lax.fori_loop(0, _NG, g, 0)

  def interp(l, rb, ifu):
    res = _RES[l]

    def g(j, _):
      px, py, pz = load_p(j)
      sx = px * np.float32(res)
      sy = py * np.float32(res)
      sz = pz * np.float32(res)
      fx = sx - sx.astype(jnp.int32).astype(jnp.float32)
      fy = sy - sy.astype(jnp.int32).astype(jnp.float32)
      fz = sz - sz.astype(jnp.int32).astype(jnp.float32)
      wx = (np.float32(1.0) - fx, fx)
      wy = (np.float32(1.0) - fy, fy)
      wz = (np.float32(1.0) - fz, fz)
      wxy = {(i, j2): wx[i] * wy[j2] for i in (0, 1) for j2 in (0, 1)}
      jb = jnp.full((_L,), j * _L, jnp.int32) + iota
      acc0 = jnp.zeros((_L,), jnp.float32)
      acc1 = jnp.zeros((_L,), jnp.float32)
      for c, (i, j2, k) in enumerate(_CORNERS):
        w = wxy[(i, j2)] * wz[k]
        i0 = jb + (c * _C)
        rv = ifu[pl.ds(c * _C + j * _L, _L)]
        c0 = lax.shift_left(rv & 3, 1)
        f0 = plsc.load_gather(rb, [i0, c0])
        f1 = plsc.load_gather(rb, [i0, c0 + 1])
        acc0 = acc0 + w * f0
        acc1 = acc1 + w * f1
      outt[2 * l, pl.ds(j * _L, _L)] = acc0
      outt[2 * l + 1, pl.ds(j * _L, _L)] = acc1
      return 0

    lax.fori_loop(0, _NG, g, 0)

  def chunk_body(ci, _):
    base = wid * per_w + ci * _C
    pltpu.sync_copy(xT.at[pl.ds(0, 3), pl.ds(base, _C)], pbuf)
    pltpu.sync_copy(xT.at[pl.ds(3, 3), pl.ds(base, _C)], outt.at[pl.ds(40, 3), :])

    ibs = (ib0, ib1)
    ifs = (if0, if1)
    rbs = (rb0, rb1)
    sems = (sem0, sem1)
    compute_idx(0, ibs[0], ifs[0])
    cps = [pltpu.async_copy(tab.at[ibs[0]], rbs[0], sems[0])]
    for l in range(1, _N_LEVELS):
      p = l & 1
      compute_idx(l, ibs[p], ifs[p])
      cps.append(pltpu.async_copy(tab.at[ibs[p]], rbs[p], sems[p]))
      cps[l - 1].wait()
      interp(l - 1, rbs[(l - 1) & 1], ifs[(l - 1) & 1])
    cps[_N_LEVELS - 1].wait()
    interp(_N_LEVELS - 1, rbs[(_N_LEVELS - 1) & 1], ifs[(_N_LEVELS - 1) & 1])

    pltpu.sync_copy(outt, encT.at[:, pl.ds(base, _C)])
    return 0

  lax.fori_loop(0, nchunk, chunk_body, 0)


def _encode_sc(xT, tab, n):
  mesh = plsc.VectorSubcoreMesh(
      core_axis_name="c", subcore_axis_name="s", num_cores=_NC, num_subcores=_NS
  )
  f = pl.kernel(
      _encode_body,
      out_type=jax.ShapeDtypeStruct((43, n), jnp.float32),
      mesh=mesh,
      scratch_types=[
          pltpu.VMEM((3, _C), jnp.float32),
          pltpu.VMEM((_NI,), jnp.int32),
          pltpu.VMEM((_NI,), jnp.int32),
          pltpu.VMEM((_NI,), jnp.int32),
          pltpu.VMEM((_NI,), jnp.int32),
          pltpu.VMEM((_NI, 8), jnp.float32),
          pltpu.VMEM((_NI, 8), jnp.float32),
          pltpu.VMEM((43, _C), jnp.float32),
          pltpu.SemaphoreType.DMA,
          pltpu.SemaphoreType.DMA,
      ],
      compiler_params=_SC_PARAMS,
  )
  return f(xT, tab)


def _mlp_body(enc_ref, w1t_ref, b1_ref, w2t_ref, b2_ref, out_ref):
  e = enc_ref[...]
  h = lax.dot_general(
      w1t_ref[...], e, (((1,), (0,)), ((), ())),
      preferred_element_type=jnp.float32,
  )
  h = jnp.maximum(h + b1_ref[...], np.float32(0.0))
  o = lax.dot_general(
      w2t_ref[...], h, (((1,), (0,)), ((), ())),
      preferred_element_type=jnp.float32,
  )
  out_ref[...] = o + b2_ref[...]


def _mlp_tc(encT, w1t, b1c, w2t, b2c, n):
  bn = 2048
  grid = (n // bn,)
  return pl.pallas_call(
      _mlp_body,
      grid=grid,
      in_specs=[
          pl.BlockSpec((43, bn), lambda i: (0, i)),
          pl.BlockSpec((64, 43), lambda i: (0, 0)),
          pl.BlockSpec((64, 1), lambda i: (0, 0)),
          pl.BlockSpec((1, 64), lambda i: (0, 0)),
          pl.BlockSpec((1, 1), lambda i: (0, 0)),
      ],
      out_specs=pl.BlockSpec((1, bn), lambda i: (0, i)),
      out_shape=jax.ShapeDtypeStruct((1, n), jnp.float32),
  )(encT, w1t, b1c, w2t, b2c)


@jax.jit
def kernel(x, table, W1, b1, W2, b2):
  n = x.shape[0]
  xT = x.T
  tabf = table.transpose(0, 2, 1).reshape(-1)
  tabi = _interleave_sc(tabf)
  encT = _encode_sc(xT, tabi, n)
  sdf = _mlp_tc(encT, W1.T, b1.reshape(64, 1), W2.T, b2.reshape(1, 1), n)
  return sdf[0]
